# baseline (speedup 1.0000x reference)
import functools

import jax
import jax.numpy as jnp
from jax import lax
from jax.experimental import pallas as pl
from jax.experimental.pallas import tpu as pltpu
from jax.experimental.pallas import tpu_sc as plsc


def _make_sc_copy(m, n, l, d, dtype):
    mesh = plsc.VectorSubcoreMesh(core_axis_name="c", subcore_axis_name="s")
    nw = mesh.num_cores * mesh.num_subcores
    assert m == nw
    chunk = 256 * d
    nchunks = (n * d) // chunk
    offset = l // 2 + l % 2

    @functools.partial(
        pl.kernel,
        out_type=jax.ShapeDtypeStruct((m * n * d,), dtype),
        mesh=mesh,
        scratch_types=[
            pltpu.VMEM((chunk,), dtype),
            pltpu.VMEM((chunk,), dtype),
            pltpu.SemaphoreType.DMA,
            pltpu.SemaphoreType.DMA,
            pltpu.SemaphoreType.DMA,
            pltpu.SemaphoreType.DMA,
        ],
    )
    def sc_copy(table, out, buf0, buf1, rs0, rs1, ws0, ws1):
        wid = lax.axis_index("s") * mesh.num_cores + lax.axis_index("c")
        rbase = (offset - wid) * d
        wbase = wid * (n * d)
        bufs = (buf0, buf1)
        rsem = (rs0, rs1)
        wsem = (ws0, ws1)
        writes = [None, None]
        for c in range(nchunks):
            b = c & 1
            if writes[b] is not None:
                writes[b].wait()
            pltpu.async_copy(
                table.at[pl.ds(rbase + c * chunk, chunk)], bufs[b], rsem[b]
            ).wait()
            wr = pltpu.make_async_copy(
                bufs[b], out.at[pl.ds(wbase + c * chunk, chunk)], wsem[b]
            )
            wr.start()
            writes[b] = wr
        writes[0].wait()
        writes[1].wait()

    return sc_copy


def kernel(q, k, embed_weight):
    m = q.shape[0]
    n = k.shape[0]
    l, d = embed_weight.shape
    flat = _make_sc_copy(m, n, l, d, embed_weight.dtype)(embed_weight.reshape(-1))
    return flat.reshape(m, n, d)


# speedup vs baseline: 1.7220x; 1.7220x over previous
import functools

import jax
import jax.numpy as jnp
from jax import lax
from jax.experimental import pallas as pl
from jax.experimental.pallas import tpu as pltpu
from jax.experimental.pallas import tpu_sc as plsc


def _make_sc_copy(m, n, l, d, dtype):
    mesh = plsc.VectorSubcoreMesh(core_axis_name="c", subcore_axis_name="s")
    nw = mesh.num_cores * mesh.num_subcores
    assert m == nw
    offset = l // 2 + l % 2
    piece = (l * d) // mesh.num_subcores

    @functools.partial(
        pl.kernel,
        out_type=jax.ShapeDtypeStruct((m * n * d,), dtype),
        mesh=mesh,
        scratch_types=[
            pltpu.VMEM_SHARED((l * d,), dtype),
            pltpu.SemaphoreType.DMA,
            pltpu.SemaphoreType.DMA,
        ],
    )
    def sc_copy(table, out, shared, ssem, wsem):
        sid = lax.axis_index("s")
        wid = sid * mesh.num_cores + lax.axis_index("c")
        pltpu.async_copy(
            table.at[pl.ds(sid * piece, piece)],
            shared.at[pl.ds(sid * piece, piece)],
            ssem,
        ).wait()
        plsc.subcore_barrier()
        pltpu.async_copy(
            shared.at[pl.ds((offset - wid) * d, n * d)],
            out.at[pl.ds(wid * (n * d), n * d)],
            wsem,
        ).wait()

    return sc_copy


def kernel(q, k, embed_weight):
    m = q.shape[0]
    n = k.shape[0]
    l, d = embed_weight.shape
    flat = _make_sc_copy(m, n, l, d, embed_weight.dtype)(embed_weight.reshape(-1))
    return flat.reshape(m, n, d)
